# baseline (speedup 1.0000x reference)
import jax
import jax.numpy as jnp
from jax import lax
from jax.experimental import pallas as pl
from jax.experimental.pallas import tpu as pltpu
from jax.experimental.pallas import tpu_sc as plsc

N = 16384
B = 16
A = 100
S = 8
ND = 256
GD = 256
HD = 256
NEG = -1000000000.0

NC = 2
NS = 16
NW = NC * NS
AP = 128
AH = A // 2
P2 = B * AP
PPT = P2 // NW
RPT = PPT * S
CROWS = 128
CPAIRS = CROWS // S
IR = P2 * S // 128
RPB = IR // B
DV = ND // 16
CHUNKS = (CROWS, CROWS, CROWS, (AH - 3 * CPAIRS) * S)


def _prep_body(batch_ref, sets_ref, p_ref, pt_ref, rows_ref, w_ref):
    bt = batch_ref[...]
    counts = [jnp.sum(jnp.where(bt == b, 1, 0)) for b in range(B)]
    rowid = lax.broadcasted_iota(jnp.int32, (IR, 1), 0)
    cnt_col = jnp.zeros((IR, 1), jnp.int32)
    off_col = jnp.zeros((IR, 1), jnp.int32)
    offs = jnp.int32(0)
    for b in range(B):
        inb = (rowid >= b * RPB) & (rowid < (b + 1) * RPB)
        cnt_col = cnt_col + jnp.where(inb, counts[b], 0)
        off_col = off_col + jnp.where(inb, offs, 0)
        offs = offs + counts[b]
    sets = sets_ref[...]
    vf = jnp.where(sets < cnt_col, 1.0, 0.0)
    rows_ref[...] = jnp.clip(sets + off_col, 0, N - 1)
    cnt = lax.dot(vf, p_ref[...], preferred_element_type=jnp.float32)
    inv = 1.0 / jnp.maximum(cnt, 1.0)
    w_ref[...] = vf * lax.dot(inv, pt_ref[...],
                              preferred_element_type=jnp.float32)


def _prep(batch2d, sets2d, pool, pool_t):
    return pl.pallas_call(
        _prep_body,
        out_shape=(
            jax.ShapeDtypeStruct((IR, 128), jnp.int32),
            jax.ShapeDtypeStruct((IR, 128), jnp.float32),
        ),
    )(batch2d, sets2d, pool, pool_t)


def _sc_body(nf_hbm, batch_hbm, sets_hbm, out_hbm, batch_v, sets_v, idx_v,
             w_v, rows_v, out_v, sem0, sem1):
    wid = lax.axis_index("s") * NC + lax.axis_index("c")
    b = wid // 2
    block8 = (wid // 2) * 8
    sub = (wid % 2) * 4
    pltpu.sync_copy(batch_hbm, batch_v)
    pltpu.sync_copy(sets_hbm.at[pl.ds(block8, 8)], sets_v)

    def hist(i, carry):
        lt, eq = carry
        for u in range(8):
            v = batch_v[pl.ds(i * 128 + u * 16, 16)]
            lt = lt + jnp.where(v < b, 1, 0)
            eq = eq + jnp.where(v == b, 1, 0)
        return lt, eq

    z16 = jnp.zeros((16,), jnp.int32)
    lt, eq = lax.fori_loop(0, N // 128, hist, (z16, z16))
    offs = jnp.sum(lt)
    cnt_b = jnp.sum(eq)

    lane = lax.iota(jnp.int32, 16)
    for r in range(4):
        for k in range(8):
            sv = sets_v[sub + r, pl.ds(k * 16, 16)]
            vf = jnp.where(sv < cnt_b, 1.0, 0.0)
            idx_v[r, pl.ds(k * 16, 16)] = jnp.clip(sv + offs, 0, N - 1)
            cs = plsc.cumsum(vf)
            c0 = cs[7]
            c1 = cs[15] - cs[7]
            cvec = jnp.where(lane < 8, c0, c1)
            w_v[r, pl.ds(k * 16, 16)] = vf / jnp.maximum(cvec, 1.0)

    sems = (sem0, sem1)
    descs = []

    def fire(c):
        buf = c % 2
        nrow = CHUNKS[c]
        descs.append(
            pltpu.async_copy(nf_hbm.at[idx_v.at[c, pl.ds(0, nrow)]],
                             rows_v.at[buf, pl.ds(0, nrow)],
                             sems[buf]))

    fire(0)
    for c in range(len(CHUNKS)):
        if c + 1 < len(CHUNKS):
            fire(c + 1)
        descs[c].wait()
        buf = c % 2

        def body(jj, carry, buf=buf, c=c):
            wvec = w_v[c, pl.ds(jj * 16, 16)]
            for half in range(2):
                rbase = (jj * 2 + half) * S
                acc = [jnp.zeros((16,), jnp.float32) for _ in range(DV)]
                for s in range(S):
                    w = wvec[half * S + s]
                    for d in range(DV):
                        acc[d] = acc[d] + w * rows_v[buf, rbase + s,
                                                     pl.ds(d * 16, 16)]
                p = c * CPAIRS + jj * 2 + half
                for d in range(DV):
                    out_v[p, pl.ds(d * 16, 16)] = acc[d]
            return carry

        lax.fori_loop(0, CHUNKS[c] // (2 * S), body, jnp.int32(0))
    pltpu.sync_copy(out_v, out_hbm.at[pl.ds(wid * PPT, PPT)])


def _make_sc_pool():
    return pl.kernel(
        _sc_body,
        out_type=jax.ShapeDtypeStruct((P2, ND), jnp.float32),
        mesh=plsc.VectorSubcoreMesh(core_axis_name="c",
                                    subcore_axis_name="s",
                                    num_cores=NC, num_subcores=NS),
        compiler_params=pltpu.CompilerParams(use_tc_tiling_on_sc=True,
                                             needs_layout_passes=False),
        scratch_types=[
            pltpu.VMEM((N,), jnp.int32),
            pltpu.VMEM((8, 128), jnp.int32),
            pltpu.VMEM((4, 128), jnp.int32),
            pltpu.VMEM((4, 128), jnp.float32),
            pltpu.VMEM((2, CROWS, ND), jnp.float32),
            pltpu.VMEM((PPT, ND), jnp.float32),
            pltpu.SemaphoreType.DMA,
            pltpu.SemaphoreType.DMA,
        ],
    )


def _mlp_body(se_ref, g_ref, w1g_ref, w1n_ref, b1_ref, w2_ref, b2_ref,
              m_ref, out_ref):
    b = pl.program_id(0)
    se = se_ref[...]
    h = lax.dot(se, w1n_ref[...], preferred_element_type=jnp.float32)
    gwall = lax.dot(g_ref[...], w1g_ref[...],
                    preferred_element_type=jnp.float32)
    brow = lax.broadcasted_iota(jnp.int32, (B, 1), 0) == b
    gw = jnp.sum(jnp.where(brow, gwall, 0.0), axis=0, keepdims=True)
    h = jnp.maximum(h + gw + b1_ref[...], 0.0)
    val = jnp.sum(h * w2_ref[...], axis=1) + b2_ref[0, :]
    mrow = jnp.sum(jnp.where(brow, m_ref[...], 0.0), axis=0)
    out_ref[0, 0, :] = jnp.where(mrow > 0, val, NEG)


def _mlp(sep_emb, gfeat, w1g, w1n, b1row, w2row, b2row, maskf):
    out = pl.pallas_call(
        _mlp_body,
        grid=(B,),
        in_specs=[
            pl.BlockSpec((AP, ND), lambda b: (b, 0)),
            pl.BlockSpec((B, GD), lambda b: (0, 0)),
            pl.BlockSpec((GD, HD), lambda b: (0, 0)),
            pl.BlockSpec((ND, HD), lambda b: (0, 0)),
            pl.BlockSpec((1, HD), lambda b: (0, 0)),
            pl.BlockSpec((1, HD), lambda b: (0, 0)),
            pl.BlockSpec((1, AP), lambda b: (0, 0)),
            pl.BlockSpec((B, AP), lambda b: (0, 0)),
        ],
        out_specs=pl.BlockSpec((1, 1, AP), lambda b: (b, 0, 0)),
        out_shape=jax.ShapeDtypeStruct((B, 1, AP), jnp.float32),
    )(sep_emb, gfeat, w1g, w1n, b1row, w2row, b2row, maskf)
    return out.reshape(B, AP)


def _interleave(x, pad_value):
    padshape = (B, PPT - AH) + x.shape[2:]
    padv = jnp.full(padshape, pad_value, x.dtype)
    return jnp.concatenate(
        [x[:, :AH], padv, x[:, AH:], padv], axis=1)


def kernel(node_features, global_features, cube_mask, batch, sep_cube_sets,
           sep_mask, W1, b1, W2, b2):
    del cube_mask
    batch1d = batch.astype(jnp.int32)
    sets_p = _interleave(sep_cube_sets.astype(jnp.int32), 0)
    sets2d = sets_p.reshape(IR, 128)

    sep_emb = _make_sc_pool()(node_features, batch1d, sets2d)

    w1g = W1[:GD, :]
    w1n = W1[GD:, :]
    b1row = b1[None, :]
    w2row = W2.reshape(1, HD)
    b2row = jnp.broadcast_to(b2.reshape(1, 1), (1, AP))
    maskf = _interleave(sep_mask.astype(jnp.float32), 0.0)

    lg = _mlp(sep_emb, global_features, w1g, w1n, b1row, w2row, b2row,
              maskf)
    return jnp.concatenate([lg[:, :AH], lg[:, PPT:PPT + AH]], axis=1)


# speedup vs baseline: 1.1153x; 1.1153x over previous
import jax
import jax.numpy as jnp
from jax import lax
from jax.experimental import pallas as pl
from jax.experimental.pallas import tpu as pltpu
from jax.experimental.pallas import tpu_sc as plsc

N = 16384
B = 16
A = 100
S = 8
ND = 256
GD = 256
HD = 256
NEG = -1000000000.0

NC = 2
NS = 16
NW = NC * NS
AP = 128
AH = A // 2
P2 = B * AP
PPT = P2 // NW
RPT = PPT * S
CROWS = 128
CPAIRS = CROWS // S
IR = P2 * S // 128
RPB = IR // B
DV = ND // 16
CHUNKS = (CROWS, CROWS, CROWS, (AH - 3 * CPAIRS) * S)


def _prep_body(batch_ref, sets_ref, p_ref, pt_ref, rows_ref, w_ref):
    bt = batch_ref[...]
    counts = [jnp.sum(jnp.where(bt == b, 1, 0)) for b in range(B)]
    rowid = lax.broadcasted_iota(jnp.int32, (IR, 1), 0)
    cnt_col = jnp.zeros((IR, 1), jnp.int32)
    off_col = jnp.zeros((IR, 1), jnp.int32)
    offs = jnp.int32(0)
    for b in range(B):
        inb = (rowid >= b * RPB) & (rowid < (b + 1) * RPB)
        cnt_col = cnt_col + jnp.where(inb, counts[b], 0)
        off_col = off_col + jnp.where(inb, offs, 0)
        offs = offs + counts[b]
    sets = sets_ref[...]
    vf = jnp.where(sets < cnt_col, 1.0, 0.0)
    rows_ref[...] = jnp.clip(sets + off_col, 0, N - 1)
    cnt = lax.dot(vf, p_ref[...], preferred_element_type=jnp.float32)
    inv = 1.0 / jnp.maximum(cnt, 1.0)
    w_ref[...] = vf * lax.dot(inv, pt_ref[...],
                              preferred_element_type=jnp.float32)


def _prep(batch2d, sets2d, pool, pool_t):
    return pl.pallas_call(
        _prep_body,
        out_shape=(
            jax.ShapeDtypeStruct((IR, 128), jnp.int32),
            jax.ShapeDtypeStruct((IR, 128), jnp.float32),
        ),
    )(batch2d, sets2d, pool, pool_t)


def _sc_body(nf_hbm, idx_hbm, w_hbm, out_hbm, idx_v, w_v, rows_v, out_v,
             sem0, sem1):
    wid = lax.axis_index("s") * NC + lax.axis_index("c")
    block8 = (wid // 2) * 8
    sub = (wid % 2) * 4
    pltpu.sync_copy(idx_hbm.at[pl.ds(block8, 8)], idx_v)
    pltpu.sync_copy(w_hbm.at[pl.ds(block8, 8)], w_v)
    sems = (sem0, sem1)
    descs = []

    def fire(c):
        buf = c % 2
        nrow = CHUNKS[c]
        descs.append(
            pltpu.async_copy(nf_hbm.at[idx_v.at[sub + c, pl.ds(0, nrow)]],
                             rows_v.at[buf, pl.ds(0, nrow)],
                             sems[buf]))

    fire(0)
    for c in range(len(CHUNKS)):
        if c + 1 < len(CHUNKS):
            fire(c + 1)
        descs[c].wait()
        buf = c % 2

        def body(jj, carry, buf=buf, c=c):
            wvec = w_v[sub + c, pl.ds(jj * 16, 16)]
            for half in range(2):
                rbase = (jj * 2 + half) * S
                acc = [jnp.zeros((16,), jnp.float32) for _ in range(DV)]
                for s in range(S):
                    w = wvec[half * S + s]
                    for d in range(DV):
                        acc[d] = acc[d] + w * rows_v[buf, rbase + s,
                                                     pl.ds(d * 16, 16)]
                p = c * CPAIRS + jj * 2 + half
                for d in range(DV):
                    out_v[p, pl.ds(d * 16, 16)] = acc[d]
            return carry

        lax.fori_loop(0, CHUNKS[c] // (2 * S), body, jnp.int32(0))
    pltpu.sync_copy(out_v, out_hbm.at[pl.ds(wid * PPT, PPT)])


def _make_sc_pool():
    return pl.kernel(
        _sc_body,
        out_type=jax.ShapeDtypeStruct((P2, ND), jnp.float32),
        mesh=plsc.VectorSubcoreMesh(core_axis_name="c",
                                    subcore_axis_name="s",
                                    num_cores=NC, num_subcores=NS),
        compiler_params=pltpu.CompilerParams(use_tc_tiling_on_sc=True),
        scratch_types=[
            pltpu.VMEM((8, 128), jnp.int32),
            pltpu.VMEM((8, 128), jnp.float32),
            pltpu.VMEM((2, CROWS, ND), jnp.float32),
            pltpu.VMEM((PPT, ND), jnp.float32),
            pltpu.SemaphoreType.DMA,
            pltpu.SemaphoreType.DMA,
        ],
    )


def _mlp_body(se_ref, g_ref, w1g_ref, w1n_ref, b1_ref, w2_ref, b2_ref,
              m_ref, out_ref):
    b = pl.program_id(0)
    se = se_ref[...]
    h = lax.dot(se, w1n_ref[...], preferred_element_type=jnp.float32)
    gwall = lax.dot(g_ref[...], w1g_ref[...],
                    preferred_element_type=jnp.float32)
    brow = lax.broadcasted_iota(jnp.int32, (B, 1), 0) == b
    gw = jnp.sum(jnp.where(brow, gwall, 0.0), axis=0, keepdims=True)
    h = jnp.maximum(h + gw + b1_ref[...], 0.0)
    val = jnp.sum(h * w2_ref[...], axis=1) + b2_ref[0, :]
    mrow = jnp.sum(jnp.where(brow, m_ref[...], 0.0), axis=0)
    lg = jnp.where(mrow > 0, val, NEG)
    out_ref[0, 0, 0:AH] = lax.slice(lg, (0,), (AH,))
    out_ref[0, 0, AH:A] = lax.slice(lg, (PPT,), (PPT + AH,))


def _mlp(sep_emb, gfeat, w1g, w1n, b1row, w2row, b2row, maskf):
    out = pl.pallas_call(
        _mlp_body,
        grid=(B,),
        in_specs=[
            pl.BlockSpec((AP, ND), lambda b: (b, 0)),
            pl.BlockSpec((B, GD), lambda b: (0, 0)),
            pl.BlockSpec((GD, HD), lambda b: (0, 0)),
            pl.BlockSpec((ND, HD), lambda b: (0, 0)),
            pl.BlockSpec((1, HD), lambda b: (0, 0)),
            pl.BlockSpec((1, HD), lambda b: (0, 0)),
            pl.BlockSpec((1, AP), lambda b: (0, 0)),
            pl.BlockSpec((B, AP), lambda b: (0, 0)),
        ],
        out_specs=pl.BlockSpec((1, 1, A), lambda b: (b, 0, 0)),
        out_shape=jax.ShapeDtypeStruct((B, 1, A), jnp.float32),
    )(sep_emb, gfeat, w1g, w1n, b1row, w2row, b2row, maskf)
    return out.reshape(B, A)


def _interleave(x, pad_value):
    padshape = (B, PPT - AH) + x.shape[2:]
    padv = jnp.full(padshape, pad_value, x.dtype)
    return jnp.concatenate(
        [x[:, :AH], padv, x[:, AH:], padv], axis=1)


def kernel(node_features, global_features, cube_mask, batch, sep_cube_sets,
           sep_mask, W1, b1, W2, b2):
    del cube_mask
    batch2d = batch.astype(jnp.int32).reshape(128, 128)
    sets_p = _interleave(sep_cube_sets.astype(jnp.int32), 0)
    sets2d = sets_p.reshape(IR, 128)
    pool = (jnp.arange(128, dtype=jnp.int32)[:, None] // S
            == jnp.arange(16, dtype=jnp.int32)[None, :]).astype(jnp.float32)
    pool_t = pool.T

    idx_hbm, w_hbm = _prep(batch2d, sets2d, pool, pool_t)

    sep_emb = _make_sc_pool()(node_features, idx_hbm, w_hbm)

    w1g = W1[:GD, :]
    w1n = W1[GD:, :]
    b1row = b1[None, :]
    w2row = W2.reshape(1, HD)
    b2row = jnp.broadcast_to(b2.reshape(1, 1), (1, AP))
    maskf = _interleave(sep_mask.astype(jnp.float32), 0.0)

    return _mlp(sep_emb, global_features, w1g, w1n, b1row, w2row, b2row,
                maskf)
